# baseline (speedup 1.0000x reference)
import functools

import jax
import jax.numpy as jnp
from jax import lax
from jax.experimental import pallas as pl
from jax.experimental.pallas import tpu as pltpu
from jax.experimental.pallas import tpu_sc as plsc

USER_N = 25000
NN = 50000
EDGES = 800000
DD = 64
BATCH = 16384

NC = 2
NS = 16
HALF = DD // 2

CH = 128
TOT_E = 2 * EDGES
CPS = 16
NSC = 49
NK = NSC * CPS
PADE = NS * NK * CH
SPAD = 50048
ROWS_PER_SUB = SPAD // NS
ZROWS = 136


def _lane_bcast(vv, e2):
    return lax.gather(
        vv, jnp.full((16, 1), e2, jnp.int32),
        lax.GatherDimensionNumbers(offset_dims=(),
                                   collapsed_slice_dims=(0,),
                                   start_index_map=(0,)),
        (1,), mode=lax.GatherScatterMode.PROMISE_IN_BOUNDS)


WB = 136
WTAIL = 88


def _sc_spmm_body(row_hbm, col_hbm, val_hbm, t4_hbm, bias_hbm, fe_hbm,
                  srow, scol, sval, gidx, rbuf, zbuf, bias_v, acc, gsem, ssem):
    c = lax.axis_index("c")
    s = lax.axis_index("s")

    def _zrow(i, carry):
        zbuf[i, pl.ds(0, 16)] = jnp.zeros((16,), jnp.float32)
        zbuf[i, pl.ds(16, 16)] = jnp.zeros((16,), jnp.float32)
        return carry

    lax.fori_loop(0, ZROWS, _zrow, 0)

    def _zcopy(i, carry):
        pltpu.sync_copy(zbuf, acc.at[pl.ds(s * ROWS_PER_SUB + i * ZROWS, ZROWS)])
        return carry

    lax.fori_loop(0, ROWS_PER_SUB // ZROWS, _zcopy, 0)
    plsc.subcore_barrier()

    crow0 = s * NK

    def _load_sc(sc):
        r0 = crow0 + sc * CPS
        pltpu.sync_copy(row_hbm.at[pl.ds(r0, CPS)], srow)
        pltpu.sync_copy(col_hbm.at[pl.ds(r0, CPS)], scol)
        pltpu.sync_copy(val_hbm.at[pl.ds(r0, CPS)], sval)

    def _prep_and_fire(kn):
        j = lax.rem(kn, CPS)
        b = lax.rem(kn, 3)

        def _g(g, cc):
            cv = scol[j, pl.ds(g * 16, 16)]
            gidx[b, pl.ds(g * 16, 16)] = cv + c
            return cc

        lax.fori_loop(0, CH // 16, _g, 0, unroll=True)
        pltpu.async_copy(t4_hbm.at[gidx.at[b]], rbuf.at[b], gsem.at[b])

    def _scatter_wait(b):
        pltpu.make_async_copy(rbuf.at[b], acc.at[srow.at[0]], ssem.at[b]).wait()

    _load_sc(0)
    _prep_and_fire(0)

    def _iter(k, carry):
        b = lax.rem(k, 3)
        j = lax.rem(k, CPS)
        kn = k + 1
        jn = lax.rem(kn, CPS)
        bn = lax.rem(kn, 3)

        @pl.when(jnp.logical_and(kn < NK, jn != 0))
        def _fire_ahead():
            @pl.when(j >= 2)
            def _drain():
                _scatter_wait(bn)

            _prep_and_fire(kn)

        pltpu.make_async_copy(t4_hbm.at[gidx.at[b]], rbuf.at[b], gsem.at[b]).wait()

        def _scale(g, cc):
            vv = sval[j, pl.ds(g * 16, 16)]
            for e2 in range(16):
                bv = _lane_bcast(vv, e2)
                e = g * 16 + e2
                r0 = rbuf[b, e, pl.ds(0, 16)]
                rbuf[b, e, pl.ds(0, 16)] = r0 * bv
                r1 = rbuf[b, e, pl.ds(16, 16)]
                rbuf[b, e, pl.ds(16, 16)] = r1 * bv
            return cc

        lax.fori_loop(0, CH // 16, _scale, 0)

        pltpu.async_copy(rbuf.at[b], acc.at[srow.at[j]], ssem.at[b], add=True)

        @pl.when(jnp.logical_and(kn < NK, jn == 0))
        def _boundary():
            _scatter_wait(b)
            _scatter_wait(lax.rem(k + 2, 3))
            _scatter_wait(bn)
            _load_sc(lax.div(kn, CPS))
            _prep_and_fire(kn)

        return carry

    lax.fori_loop(0, NK, _iter, 0)
    _scatter_wait(0)
    _scatter_wait(1)
    _scatter_wait(2)
    plsc.subcore_barrier()

    pltpu.sync_copy(bias_hbm.at[c], bias_v)
    b0 = bias_v[pl.ds(0, 16)]
    b1 = bias_v[pl.ds(16, 16)]
    r0 = s * ROWS_PER_SUB
    col0 = 2 * DD // 2 + HALF * c

    def _wchunk(base, nr_rows):
        pltpu.sync_copy(acc.at[pl.ds(base, nr_rows)], zbuf.at[pl.ds(0, nr_rows)])

        def _relu(r, cc):
            x0 = zbuf[r, pl.ds(0, 16)]
            zbuf[r, pl.ds(0, 16)] = jnp.maximum(x0 + b0, 0.0)
            x1 = zbuf[r, pl.ds(16, 16)]
            zbuf[r, pl.ds(16, 16)] = jnp.maximum(x1 + b1, 0.0)
            return cc

        lax.fori_loop(0, nr_rows, _relu, 0)
        pltpu.sync_copy(zbuf.at[pl.ds(0, nr_rows)],
                        fe_hbm.at[pl.ds(base, nr_rows), pl.ds(col0, HALF)])

    nfull = jnp.where(s == NS - 1, (ROWS_PER_SUB - 48 - WTAIL) // WB,
                      ROWS_PER_SUB // WB)

    def _wloop(i, carry):
        _wchunk(r0 + i * WB, WB)
        return carry

    lax.fori_loop(0, nfull, _wloop, 0)

    @pl.when(s == NS - 1)
    def _wtail():
        _wchunk(r0 + ((ROWS_PER_SUB - 48 - WTAIL) // WB) * WB, WTAIL)


_sc_spmm = pl.kernel(
    _sc_spmm_body,
    out_type=jax.ShapeDtypeStruct((NN, 2 * DD), jnp.float32),
    mesh=plsc.VectorSubcoreMesh(core_axis_name="c", subcore_axis_name="s"),
    scratch_types=[
        pltpu.VMEM((CPS, CH), jnp.int32),
        pltpu.VMEM((CPS, CH), jnp.int32),
        pltpu.VMEM((CPS, CH), jnp.float32),
        pltpu.VMEM((3, CH), jnp.int32),
        pltpu.VMEM((3, CH, HALF), jnp.float32),
        pltpu.VMEM((ZROWS, HALF), jnp.float32),
        pltpu.VMEM((HALF,), jnp.float32),
        pltpu.VMEM_SHARED((SPAD, HALF), jnp.float32),
        pltpu.SemaphoreType.DMA((3,)),
        pltpu.SemaphoreType.DMA((3,)),
    ],
    compiler_params=pltpu.CompilerParams(use_tc_tiling_on_sc=False),
)


ROWS_PER_W = BATCH // (NC * NS)
GCH = 128


def _sc_gather_body(fe_hbm, uidx_hbm, iidx_hbm, ue_hbm, ie_hbm,
                    idx_v, gbuf, sem):
    c = lax.axis_index("c")
    s = lax.axis_index("s")
    wid = s * NC + c

    def _table(idx_hbm, out_hbm):
        def _ch(j, carry):
            base = wid * ROWS_PER_W + j * GCH
            pltpu.sync_copy(idx_hbm.at[pl.ds(base, GCH)], idx_v)
            pltpu.async_copy(fe_hbm.at[idx_v], gbuf, sem).wait()
            pltpu.sync_copy(gbuf, out_hbm.at[pl.ds(base, GCH)])
            return carry

        lax.fori_loop(0, ROWS_PER_W // GCH, _ch, 0)

    _table(uidx_hbm, ue_hbm)
    _table(iidx_hbm, ie_hbm)


_sc_gather = pl.kernel(
    _sc_gather_body,
    out_type=[
        jax.ShapeDtypeStruct((BATCH, 2 * DD), jnp.float32),
        jax.ShapeDtypeStruct((BATCH, 2 * DD), jnp.float32),
    ],
    mesh=plsc.VectorSubcoreMesh(core_axis_name="c", subcore_axis_name="s"),
    scratch_types=[
        pltpu.VMEM((GCH,), jnp.int32),
        pltpu.VMEM((GCH, 2 * DD), jnp.float32),
        pltpu.SemaphoreType.DMA,
    ],
)


RB = 2000
NB = NN // RB


def _tc_pre_body(f_ref, wl_ref, wi_ref, wl1_ref, wi1_ref, t_ref):
    x = f_ref[...]
    x2 = x * x
    g = (jnp.dot(x, wl_ref[...], preferred_element_type=jnp.float32,
                  precision=lax.Precision.HIGHEST)
         + jnp.dot(x2, wi_ref[...], preferred_element_type=jnp.float32,
                  precision=lax.Precision.HIGHEST))
    h = (jnp.dot(x, wl1_ref[...], preferred_element_type=jnp.float32,
                  precision=lax.Precision.HIGHEST)
         + jnp.dot(x2, wi1_ref[...], preferred_element_type=jnp.float32,
                  precision=lax.Precision.HIGHEST))
    t_ref[...] = jnp.concatenate([g, h], axis=1)


_tc_pre = pl.pallas_call(
    _tc_pre_body,
    grid=(NB,),
    in_specs=[
        pl.BlockSpec((RB, DD), lambda i: (i, 0)),
        pl.BlockSpec((DD, DD), lambda i: (0, 0)),
        pl.BlockSpec((DD, DD), lambda i: (0, 0)),
        pl.BlockSpec((DD, DD), lambda i: (0, 0)),
        pl.BlockSpec((DD, DD), lambda i: (0, 0)),
    ],
    out_specs=pl.BlockSpec((RB, 2 * DD), lambda i: (i, 0)),
    out_shape=jax.ShapeDtypeStruct((NN, 2 * DD), jnp.float32),
)


FRB = 2000
FNB = NN // FRB


def _tc_fin_body(fe_ref, f_ref, out_ref):
    out_ref[...] = jnp.concatenate([f_ref[...], fe_ref[...][:, DD:]], axis=1)


_tc_fin = pl.pallas_call(
    _tc_fin_body,
    grid=(FNB,),
    in_specs=[
        pl.BlockSpec((FRB, 2 * DD), lambda i: (i, 0)),
        pl.BlockSpec((FRB, DD), lambda i: (i, 0)),
    ],
    out_specs=pl.BlockSpec((FRB, 2 * DD), lambda i: (i, 0)),
    out_shape=jax.ShapeDtypeStruct((NN, 2 * DD), jnp.float32),
    input_output_aliases={0: 0},
)


HB = 2048
HNB = BATCH // HB


def _tc_head_body(u_ref, i_ref, w1u_ref, w1i_ref, b1_ref, w2_ref, b2_ref,
                  w3_ref, b3_ref, out_ref):
    u = u_ref[...]
    it = i_ref[...]
    h = (jnp.dot(u, w1u_ref[...], preferred_element_type=jnp.float32,
                  precision=lax.Precision.HIGHEST)
         + jnp.dot(it, w1i_ref[...], preferred_element_type=jnp.float32,
                  precision=lax.Precision.HIGHEST)
         + b1_ref[...])
    h = jnp.maximum(h, 0.0)
    h2 = jnp.dot(h, w2_ref[...], preferred_element_type=jnp.float32,
                  precision=lax.Precision.HIGHEST) + b2_ref[...]
    p = jnp.sum(h2 * w3_ref[...], axis=1, keepdims=True) + b3_ref[...]
    out_ref[...] = p


_tc_head = pl.pallas_call(
    _tc_head_body,
    grid=(HNB,),
    in_specs=[
        pl.BlockSpec((HB, 2 * DD), lambda i: (i, 0)),
        pl.BlockSpec((HB, 2 * DD), lambda i: (i, 0)),
        pl.BlockSpec((2 * DD, DD), lambda i: (0, 0)),
        pl.BlockSpec((2 * DD, DD), lambda i: (0, 0)),
        pl.BlockSpec((1, DD), lambda i: (0, 0)),
        pl.BlockSpec((DD, HALF), lambda i: (0, 0)),
        pl.BlockSpec((1, HALF), lambda i: (0, 0)),
        pl.BlockSpec((1, HALF), lambda i: (0, 0)),
        pl.BlockSpec((1, 1), lambda i: (0, 0)),
    ],
    out_specs=pl.BlockSpec((HB, 1), lambda i: (i, 0)),
    out_shape=jax.ShapeDtypeStruct((BATCH, 1), jnp.float32),
)


@jax.jit
def kernel(userIdx, itemIdx, L_row, L_col, L_val, L3_row, L3_col, L3_val,
           uEmbd, iEmbd, W_lin, b_lin, W_lin1, b_lin1, W_inter, b_inter,
           W_inter1, b_inter1, W1, b1, W2, b2, W3, b3):
    uidx = userIdx.astype(jnp.int32)
    iidx = (itemIdx + USER_N).astype(jnp.int32)

    F = jnp.concatenate([uEmbd, iEmbd], axis=0)
    npad = PADE - TOT_E
    pidx = jnp.arange(npad, dtype=jnp.int32) * 7 % NN
    cat_row = jnp.concatenate(
        [L_row.astype(jnp.int32), L3_row.astype(jnp.int32), pidx]).reshape(-1, CH)
    cat_col = jnp.concatenate(
        [L_col.astype(jnp.int32) * 4, L3_col.astype(jnp.int32) * 4 + 2,
         pidx * 4]).reshape(-1, CH)
    cat_val = jnp.concatenate(
        [L_val, L3_val, jnp.zeros((npad,), jnp.float32)]).reshape(-1, CH)

    t_wide = _tc_pre(F, W_lin, W_inter, W_lin1, W_inter1)
    t4 = t_wide.reshape(4 * NN, HALF)

    bsum = (b_lin + b_inter + b_lin1 + b_inter1).reshape(2, HALF)
    fe_half = _sc_spmm(cat_row, cat_col, cat_val, t4, bsum)

    final_embd = _tc_fin(fe_half, F)

    u_embd, i_embd = _sc_gather(final_embd, uidx, iidx)

    pred = _tc_head(u_embd, i_embd, W1[:2 * DD], W1[2 * DD:],
                    b1.reshape(1, DD), W2, b2.reshape(1, HALF),
                    W3.reshape(1, HALF), b3.reshape(1, 1))
    return (pred.reshape(BATCH), u_embd, i_embd, final_embd)


# speedup vs baseline: 1.7081x; 1.7081x over previous
import functools

import jax
import jax.numpy as jnp
from jax import lax
from jax.experimental import pallas as pl
from jax.experimental.pallas import tpu as pltpu
from jax.experimental.pallas import tpu_sc as plsc

USER_N = 25000
NN = 50000
EDGES = 800000
DD = 64
BATCH = 16384

NC = 2
NS = 16
HALF = DD // 2

CH = 128
TOT_E = 2 * EDGES
CPS = 16
NSC = 49
NK = NSC * CPS
PADE = NS * NK * CH
SPAD = 50048
ROWS_PER_SUB = SPAD // NS
ZROWS = 136


def _lane_bcast(vv, e2):
    return lax.gather(
        vv, jnp.full((16, 1), e2, jnp.int32),
        lax.GatherDimensionNumbers(offset_dims=(),
                                   collapsed_slice_dims=(0,),
                                   start_index_map=(0,)),
        (1,), mode=lax.GatherScatterMode.PROMISE_IN_BOUNDS)


WB = 136
WTAIL = 88


def _sc_spmm_body(row_hbm, col_hbm, val_hbm, t4_hbm, bias_hbm, fe_hbm,
                  srow, scol, sval, gidx, rbuf, zbuf, bias_v, acc, gsem, ssem):
    c = lax.axis_index("c")
    s = lax.axis_index("s")

    def _zrow(i, carry):
        zbuf[i, pl.ds(0, 16)] = jnp.zeros((16,), jnp.float32)
        zbuf[i, pl.ds(16, 16)] = jnp.zeros((16,), jnp.float32)
        return carry

    lax.fori_loop(0, ZROWS, _zrow, 0)

    def _zcopy(i, carry):
        pltpu.sync_copy(zbuf, acc.at[pl.ds(s * ROWS_PER_SUB + i * ZROWS, ZROWS)])
        return carry

    lax.fori_loop(0, ROWS_PER_SUB // ZROWS, _zcopy, 0)
    plsc.subcore_barrier()

    crow0 = s * NK

    def _load_sc(sc):
        r0 = crow0 + sc * CPS
        pltpu.sync_copy(row_hbm.at[pl.ds(r0, CPS)], srow)
        pltpu.sync_copy(col_hbm.at[pl.ds(r0, CPS)], scol)
        pltpu.sync_copy(val_hbm.at[pl.ds(r0, CPS)], sval)

    def _prep_and_fire(kn):
        j = lax.rem(kn, CPS)
        b = lax.rem(kn, 4)

        def _g(g, cc):
            cv = scol[j, pl.ds(g * 16, 16)]
            gidx[b, pl.ds(g * 16, 16)] = cv + c
            return cc

        lax.fori_loop(0, CH // 16, _g, 0, unroll=True)
        pltpu.async_copy(t4_hbm.at[gidx.at[b]], rbuf.at[b], gsem.at[b])

    def _scatter_wait(b):
        pltpu.make_async_copy(rbuf.at[b], acc.at[srow.at[0]], ssem.at[b]).wait()

    _load_sc(0)
    _prep_and_fire(0)

    def _iter(k, carry):
        b = lax.rem(k, 4)
        j = lax.rem(k, CPS)
        kn = k + 1
        jn = lax.rem(kn, CPS)
        bn = lax.rem(kn, 4)

        @pl.when(jnp.logical_and(kn < NK, jn != 0))
        def _fire_ahead():
            @pl.when(j >= 3)
            def _drain():
                _scatter_wait(bn)

            _prep_and_fire(kn)

        pltpu.make_async_copy(t4_hbm.at[gidx.at[b]], rbuf.at[b], gsem.at[b]).wait()

        def _scale(g, cc):
            vv = sval[j, pl.ds(g * 16, 16)]
            for e2 in range(16):
                bv = _lane_bcast(vv, e2)
                e = g * 16 + e2
                r0 = rbuf[b, e, pl.ds(0, 16)]
                rbuf[b, e, pl.ds(0, 16)] = r0 * bv
                r1 = rbuf[b, e, pl.ds(16, 16)]
                rbuf[b, e, pl.ds(16, 16)] = r1 * bv
            return cc

        lax.fori_loop(0, CH // 16, _scale, 0)

        pltpu.async_copy(rbuf.at[b], acc.at[srow.at[j]], ssem.at[b], add=True)

        @pl.when(jnp.logical_and(kn < NK, jn == 0))
        def _boundary():
            _scatter_wait(b)
            _scatter_wait(lax.rem(k + 3, 4))
            _scatter_wait(lax.rem(k + 2, 4))
            _scatter_wait(bn)
            _load_sc(lax.div(kn, CPS))
            _prep_and_fire(kn)

        return carry

    lax.fori_loop(0, NK, _iter, 0)
    _scatter_wait(0)
    _scatter_wait(1)
    _scatter_wait(2)
    _scatter_wait(3)
    plsc.subcore_barrier()

    pltpu.sync_copy(bias_hbm.at[c], bias_v)
    b0 = bias_v[pl.ds(0, 16)]
    b1 = bias_v[pl.ds(16, 16)]
    r0 = s * ROWS_PER_SUB
    col0 = 2 * DD // 2 + HALF * c

    def _wchunk(base, nr_rows):
        pltpu.sync_copy(acc.at[pl.ds(base, nr_rows)], zbuf.at[pl.ds(0, nr_rows)])

        def _relu(r, cc):
            x0 = zbuf[r, pl.ds(0, 16)]
            zbuf[r, pl.ds(0, 16)] = jnp.maximum(x0 + b0, 0.0)
            x1 = zbuf[r, pl.ds(16, 16)]
            zbuf[r, pl.ds(16, 16)] = jnp.maximum(x1 + b1, 0.0)
            return cc

        lax.fori_loop(0, nr_rows, _relu, 0)
        pltpu.sync_copy(zbuf.at[pl.ds(0, nr_rows)],
                        fe_hbm.at[pl.ds(base, nr_rows), pl.ds(col0, HALF)])

    nfull = jnp.where(s == NS - 1, (ROWS_PER_SUB - 48 - WTAIL) // WB,
                      ROWS_PER_SUB // WB)

    def _wloop(i, carry):
        _wchunk(r0 + i * WB, WB)
        return carry

    lax.fori_loop(0, nfull, _wloop, 0)

    @pl.when(s == NS - 1)
    def _wtail():
        _wchunk(r0 + ((ROWS_PER_SUB - 48 - WTAIL) // WB) * WB, WTAIL)


_sc_spmm = pl.kernel(
    _sc_spmm_body,
    out_type=jax.ShapeDtypeStruct((NN, 2 * DD), jnp.float32),
    mesh=plsc.VectorSubcoreMesh(core_axis_name="c", subcore_axis_name="s"),
    scratch_types=[
        pltpu.VMEM((CPS, CH), jnp.int32),
        pltpu.VMEM((CPS, CH), jnp.int32),
        pltpu.VMEM((CPS, CH), jnp.float32),
        pltpu.VMEM((4, CH), jnp.int32),
        pltpu.VMEM((4, CH, HALF), jnp.float32),
        pltpu.VMEM((ZROWS, HALF), jnp.float32),
        pltpu.VMEM((HALF,), jnp.float32),
        pltpu.VMEM_SHARED((SPAD, HALF), jnp.float32),
        pltpu.SemaphoreType.DMA((4,)),
        pltpu.SemaphoreType.DMA((4,)),
    ],
    compiler_params=pltpu.CompilerParams(use_tc_tiling_on_sc=False),
)


ROWS_PER_W = BATCH // (NC * NS)
GCH = 128


def _sc_gather_body(fe_hbm, uidx_hbm, iidx_hbm, ue_hbm, ie_hbm,
                    idx_v, gbuf, sem):
    c = lax.axis_index("c")
    s = lax.axis_index("s")
    wid = s * NC + c

    def _table(idx_hbm, out_hbm):
        def _ch(j, carry):
            base = wid * ROWS_PER_W + j * GCH
            pltpu.sync_copy(idx_hbm.at[pl.ds(base, GCH)], idx_v)
            pltpu.async_copy(fe_hbm.at[idx_v], gbuf, sem).wait()
            pltpu.sync_copy(gbuf, out_hbm.at[pl.ds(base, GCH)])
            return carry

        lax.fori_loop(0, ROWS_PER_W // GCH, _ch, 0)

    _table(uidx_hbm, ue_hbm)
    _table(iidx_hbm, ie_hbm)


_sc_gather = pl.kernel(
    _sc_gather_body,
    out_type=[
        jax.ShapeDtypeStruct((BATCH, 2 * DD), jnp.float32),
        jax.ShapeDtypeStruct((BATCH, 2 * DD), jnp.float32),
    ],
    mesh=plsc.VectorSubcoreMesh(core_axis_name="c", subcore_axis_name="s"),
    scratch_types=[
        pltpu.VMEM((GCH,), jnp.int32),
        pltpu.VMEM((GCH, 2 * DD), jnp.float32),
        pltpu.SemaphoreType.DMA,
    ],
)


RB = 2000
NB = NN // RB


def _tc_pre_body(f_ref, wl_ref, wi_ref, wl1_ref, wi1_ref, t_ref):
    x = f_ref[...]
    x2 = x * x
    g = (jnp.dot(x, wl_ref[...], preferred_element_type=jnp.float32,
                  precision=lax.Precision.HIGHEST)
         + jnp.dot(x2, wi_ref[...], preferred_element_type=jnp.float32,
                  precision=lax.Precision.HIGHEST))
    h = (jnp.dot(x, wl1_ref[...], preferred_element_type=jnp.float32,
                  precision=lax.Precision.HIGHEST)
         + jnp.dot(x2, wi1_ref[...], preferred_element_type=jnp.float32,
                  precision=lax.Precision.HIGHEST))
    t_ref[...] = jnp.concatenate([g, h], axis=1)


_tc_pre = pl.pallas_call(
    _tc_pre_body,
    grid=(NB,),
    in_specs=[
        pl.BlockSpec((RB, DD), lambda i: (i, 0)),
        pl.BlockSpec((DD, DD), lambda i: (0, 0)),
        pl.BlockSpec((DD, DD), lambda i: (0, 0)),
        pl.BlockSpec((DD, DD), lambda i: (0, 0)),
        pl.BlockSpec((DD, DD), lambda i: (0, 0)),
    ],
    out_specs=pl.BlockSpec((RB, 2 * DD), lambda i: (i, 0)),
    out_shape=jax.ShapeDtypeStruct((NN, 2 * DD), jnp.float32),
)


FRB = 2000
FNB = NN // FRB


def _tc_fin_body(fe_ref, f_ref, out_ref):
    out_ref[...] = jnp.concatenate([f_ref[...], fe_ref[...][:, DD:]], axis=1)


_tc_fin = pl.pallas_call(
    _tc_fin_body,
    grid=(FNB,),
    in_specs=[
        pl.BlockSpec((FRB, 2 * DD), lambda i: (i, 0)),
        pl.BlockSpec((FRB, DD), lambda i: (i, 0)),
    ],
    out_specs=pl.BlockSpec((FRB, 2 * DD), lambda i: (i, 0)),
    out_shape=jax.ShapeDtypeStruct((NN, 2 * DD), jnp.float32),
    input_output_aliases={0: 0},
)


HB = 2048
HNB = BATCH // HB


def _tc_head_body(u_ref, i_ref, w1u_ref, w1i_ref, b1_ref, w2_ref, b2_ref,
                  w3_ref, b3_ref, out_ref):
    u = u_ref[...]
    it = i_ref[...]
    h = (jnp.dot(u, w1u_ref[...], preferred_element_type=jnp.float32,
                  precision=lax.Precision.HIGHEST)
         + jnp.dot(it, w1i_ref[...], preferred_element_type=jnp.float32,
                  precision=lax.Precision.HIGHEST)
         + b1_ref[...])
    h = jnp.maximum(h, 0.0)
    h2 = jnp.dot(h, w2_ref[...], preferred_element_type=jnp.float32,
                  precision=lax.Precision.HIGHEST) + b2_ref[...]
    p = jnp.sum(h2 * w3_ref[...], axis=1, keepdims=True) + b3_ref[...]
    out_ref[...] = p


_tc_head = pl.pallas_call(
    _tc_head_body,
    grid=(HNB,),
    in_specs=[
        pl.BlockSpec((HB, 2 * DD), lambda i: (i, 0)),
        pl.BlockSpec((HB, 2 * DD), lambda i: (i, 0)),
        pl.BlockSpec((2 * DD, DD), lambda i: (0, 0)),
        pl.BlockSpec((2 * DD, DD), lambda i: (0, 0)),
        pl.BlockSpec((1, DD), lambda i: (0, 0)),
        pl.BlockSpec((DD, HALF), lambda i: (0, 0)),
        pl.BlockSpec((1, HALF), lambda i: (0, 0)),
        pl.BlockSpec((1, HALF), lambda i: (0, 0)),
        pl.BlockSpec((1, 1), lambda i: (0, 0)),
    ],
    out_specs=pl.BlockSpec((HB, 1), lambda i: (i, 0)),
    out_shape=jax.ShapeDtypeStruct((BATCH, 1), jnp.float32),
)


@jax.jit
def kernel(userIdx, itemIdx, L_row, L_col, L_val, L3_row, L3_col, L3_val,
           uEmbd, iEmbd, W_lin, b_lin, W_lin1, b_lin1, W_inter, b_inter,
           W_inter1, b_inter1, W1, b1, W2, b2, W3, b3):
    uidx = userIdx.astype(jnp.int32)
    iidx = (itemIdx + USER_N).astype(jnp.int32)

    F = jnp.concatenate([uEmbd, iEmbd], axis=0)
    npad = PADE - TOT_E
    pidx = jnp.arange(npad, dtype=jnp.int32) * 7 % NN
    cat_row = jnp.concatenate(
        [L_row.astype(jnp.int32), L3_row.astype(jnp.int32), pidx]).reshape(-1, CH)
    cat_col = jnp.concatenate(
        [L_col.astype(jnp.int32) * 4, L3_col.astype(jnp.int32) * 4 + 2,
         pidx * 4]).reshape(-1, CH)
    cat_val = jnp.concatenate(
        [L_val, L3_val, jnp.zeros((npad,), jnp.float32)]).reshape(-1, CH)

    t_wide = _tc_pre(F, W_lin, W_inter, W_lin1, W_inter1)
    t4 = t_wide.reshape(4 * NN, HALF)

    bsum = (b_lin + b_inter + b_lin1 + b_inter1).reshape(2, HALF)
    fe_half = _sc_spmm(cat_row, cat_col, cat_val, t4, bsum)

    final_embd = _tc_fin(fe_half, F)

    u_embd, i_embd = _sc_gather(final_embd, uidx, iidx)

    pred = _tc_head(u_embd, i_embd, W1[:2 * DD], W1[2 * DD:],
                    b1.reshape(1, DD), W2, b2.reshape(1, HALF),
                    W3.reshape(1, HALF), b3.reshape(1, 1))
    return (pred.reshape(BATCH), u_embd, i_embd, final_embd)
